# baseline (speedup 1.0000x reference)
import functools

import jax
import jax.numpy as jnp
from jax import lax
from jax.experimental import pallas as pl
from jax.experimental.pallas import tpu as pltpu
from jax.experimental.pallas import tpu_sc as plsc

N_NODES = 10000
N_EDGES = 320000
N_PAD = 10240
H = N_PAD // 2
ROW_BLK = 2560
NC = 2
NS = 16
NW = NC * NS
CHUNK = 128
NRING = 2

ECHUNKS = 160
E_PAD = NS * ECHUNKS * CHUNK
DEG_CHUNK = 128
DEG_CHUNKS = 79
DEG_E_PAD = NW * DEG_CHUNKS * DEG_CHUNK
DEG_TRASH = N_PAD - 1

ROWS_PER_TILE = N_PAD // NS
HROWS_PER_TILE = H // NS


def _sc_mesh():
  return plsc.VectorSubcoreMesh(core_axis_name="c", subcore_axis_name="s")


def _deg_kernel_body(dst_hbm, out_hbm, dst_v, ones_v, zero_v, sem, deg_sh):
  c = lax.axis_index("c")
  s = lax.axis_index("s")
  wid = s * NC + c
  pltpu.sync_copy(dst_hbm.at[wid], dst_v)

  def fill(i, _):
    ones_v[i, :] = jnp.ones((16,), jnp.float32)
    return 0
  lax.fori_loop(0, DEG_CHUNK, fill, 0)

  def zfill(i, _):
    zero_v[i, :] = jnp.zeros((16,), jnp.float32)
    return 0
  lax.fori_loop(0, ROWS_PER_TILE, zfill, 0)

  nslice = pl.ds(s * ROWS_PER_TILE, ROWS_PER_TILE)
  pltpu.sync_copy(zero_v, deg_sh.at[nslice])
  plsc.subcore_barrier()

  def body(j, _):
    pltpu.async_copy(ones_v, deg_sh.at[dst_v.at[j]], sem, add=True).wait()
    return 0
  lax.fori_loop(0, DEG_CHUNKS, body, 0)

  plsc.subcore_barrier()
  pltpu.sync_copy(deg_sh.at[nslice], zero_v)
  pltpu.sync_copy(
      zero_v,
      out_hbm.at[pl.ds(c * N_PAD + s * ROWS_PER_TILE, ROWS_PER_TILE)],
  )


@functools.cache
def _deg_kernel():
  return pl.kernel(
      _deg_kernel_body,
      out_type=jax.ShapeDtypeStruct((NC * N_PAD, 16), jnp.float32),
      mesh=_sc_mesh(),
      scratch_types=[
          pltpu.VMEM((DEG_CHUNKS, DEG_CHUNK), jnp.int32),
          pltpu.VMEM((DEG_CHUNK, 16), jnp.float32),
          pltpu.VMEM((ROWS_PER_TILE, 16), jnp.float32),
          pltpu.SemaphoreType.DMA,
          pltpu.VMEM_SHARED((N_PAD, 16), jnp.float32),
      ],
  )


def _agg_kernel_body(y_hbm, src_hbm, dst_hbm, out_hbm,
                     src_v, dst_v, rows_v, zero_v, gsem, ssem, agg_sh):
  c = lax.axis_index("c")
  s = lax.axis_index("s")
  pltpu.sync_copy(src_hbm.at[c, s], src_v)
  pltpu.sync_copy(dst_hbm.at[c, s], dst_v)

  def zfill(i, _):
    for k in range(8):
      zero_v[i, pl.ds(k * 16, 16)] = jnp.zeros((16,), jnp.float32)
    return 0
  lax.fori_loop(0, HROWS_PER_TILE // 4, zfill, 0)

  for k in range(4):
    pltpu.sync_copy(
        zero_v,
        agg_sh.at[pl.ds(s * HROWS_PER_TILE + k * (HROWS_PER_TILE // 4),
                        HROWS_PER_TILE // 4)],
    )

  @pl.when(s == NS - 1)
  def _():
    pltpu.sync_copy(zero_v.at[pl.ds(0, 8)], agg_sh.at[pl.ds(H, 8)])

  plsc.subcore_barrier()

  def body_t(t, _):
    ghs = []
    for b in range(NRING):
      j = t * NRING + b
      ghs.append(pltpu.async_copy(
          y_hbm.at[src_v.at[j]], rows_v.at[b], gsem))
    for gh in ghs:
      gh.wait()
    shs = []
    for b in range(NRING):
      j = t * NRING + b
      shs.append(pltpu.async_copy(
          rows_v.at[b], agg_sh.at[dst_v.at[j]], ssem, add=True))
    for sh in shs:
      sh.wait()
    return 0

  lax.fori_loop(0, ECHUNKS // NRING, body_t, 0)

  plsc.subcore_barrier()
  pltpu.sync_copy(
      agg_sh.at[pl.ds(s * HROWS_PER_TILE, HROWS_PER_TILE)],
      out_hbm.at[pl.ds(c * H + s * HROWS_PER_TILE, HROWS_PER_TILE)])


@functools.cache
def _agg_kernel():
  return pl.kernel(
      _agg_kernel_body,
      out_type=jax.ShapeDtypeStruct((N_PAD, 128), jnp.float32),
      mesh=_sc_mesh(),
      scratch_types=[
          pltpu.VMEM((ECHUNKS, CHUNK), jnp.int32),
          pltpu.VMEM((ECHUNKS, CHUNK), jnp.int32),
          pltpu.VMEM((NRING, CHUNK, 128), jnp.float32),
          pltpu.VMEM((HROWS_PER_TILE // 4, 128), jnp.float32),
          pltpu.SemaphoreType.DMA,
          pltpu.SemaphoreType.DMA,
          pltpu.VMEM_SHARED((H + 8, 128), jnp.float32),
      ],
  )


def _dis_from_deg(deg_ref):
  return lax.rsqrt(deg_ref[:, 0:1] + 1.0)


def _dot(a, b):
  return jnp.dot(a, b, preferred_element_type=jnp.float32,
                 precision=lax.Precision.HIGHEST)


def _enc_body(x_ref, deg_ref, w_ref, y_ref):
  dis = _dis_from_deg(deg_ref)
  y_ref[...] = dis * _dot(x_ref[...], w_ref[...])


_tc_enc = pl.pallas_call(
    _enc_body,
    grid=(N_PAD // ROW_BLK,),
    in_specs=[
        pl.BlockSpec((ROW_BLK, 128), lambda i: (i, 0)),
        pl.BlockSpec((ROW_BLK, 128), lambda i: (i, 0)),
        pl.BlockSpec((128, 128), lambda i: (0, 0)),
    ],
    out_specs=pl.BlockSpec((ROW_BLK, 128), lambda i: (i, 0)),
    out_shape=jax.ShapeDtypeStruct((N_PAD, 128), jnp.float32),
)


def _mid_body(a_ref, y_ref, deg_ref, b_ref, w_ref, o_ref):
  dis = _dis_from_deg(deg_ref)
  h = jnp.maximum(dis * (a_ref[...] + y_ref[...]) + b_ref[...], 0.0)
  o_ref[...] = dis * _dot(h, w_ref[...])


_tc_mid = pl.pallas_call(
    _mid_body,
    grid=(N_PAD // ROW_BLK,),
    in_specs=[
        pl.BlockSpec((ROW_BLK, 128), lambda i: (i, 0)),
        pl.BlockSpec((ROW_BLK, 128), lambda i: (i, 0)),
        pl.BlockSpec((ROW_BLK, 128), lambda i: (i, 0)),
        pl.BlockSpec((1, 128), lambda i: (0, 0)),
        pl.BlockSpec((128, 128), lambda i: (0, 0)),
    ],
    out_specs=pl.BlockSpec((ROW_BLK, 128), lambda i: (i, 0)),
    out_shape=jax.ShapeDtypeStruct((N_PAD, 128), jnp.float32),
)


def _out_body(a_ref, y_ref, deg_ref, b_ref, o_ref):
  dis = _dis_from_deg(deg_ref)
  o_ref[...] = jnp.maximum(dis * (a_ref[...] + y_ref[...]) + b_ref[...], 0.0)


_tc_out = pl.pallas_call(
    _out_body,
    grid=(N_PAD // ROW_BLK,),
    in_specs=[
        pl.BlockSpec((ROW_BLK, 128), lambda i: (i, 0)),
        pl.BlockSpec((ROW_BLK, 128), lambda i: (i, 0)),
        pl.BlockSpec((ROW_BLK, 128), lambda i: (i, 0)),
        pl.BlockSpec((1, 128), lambda i: (0, 0)),
    ],
    out_specs=pl.BlockSpec((ROW_BLK, 128), lambda i: (i, 0)),
    out_shape=jax.ShapeDtypeStruct((N_PAD, 128), jnp.float32),
)


@jax.jit
def kernel(x, edge_index, W_enc, b_enc, W_conv, b_conv, W_dec, b_dec):
  src = edge_index[0]
  dst = edge_index[1]

  pad = E_PAD - N_EDGES
  src_p = jnp.concatenate([src, jnp.zeros((pad,), jnp.int32)])
  dst_p = jnp.concatenate([dst, jnp.full((pad,), -1, jnp.int32)])
  trash = jnp.int32(H)
  src_all = jnp.stack([src_p, src_p]).reshape(NC, NS, ECHUNKS, CHUNK)
  deg_src = jnp.zeros_like(src_all)
  dst_cs = []
  for c in range(NC):
    in_c = (dst_p >= c * H) & (dst_p < (c + 1) * H)
    dst_cs.append(jnp.where(in_c, dst_p - c * H, trash))
  dst_all = jnp.stack(dst_cs).reshape(NC, NS, ECHUNKS, CHUNK)

  x_pad = jnp.pad(x, ((0, N_PAD - N_NODES), (0, 0)))
  w_dec = jnp.pad(W_dec, ((0, 0), (0, 128 - W_dec.shape[1])))
  b_dec_p = jnp.pad(b_dec, (0, 128 - b_dec.shape[0])).reshape(1, 128)
  b_enc_r = b_enc.reshape(1, 128)
  b_conv_r = b_conv.reshape(1, 128)

  agg_k = _agg_kernel()
  ones_t = jnp.ones((N_PAD, 128), jnp.float32)
  deg_full = agg_k(ones_t, deg_src, dst_all)

  y0 = _tc_enc(x_pad, deg_full, W_enc)
  a0 = agg_k(y0, src_all, dst_all)
  y1 = _tc_mid(a0, y0, deg_full, b_enc_r, W_conv)
  a1 = agg_k(y1, src_all, dst_all)
  y2 = _tc_mid(a1, y1, deg_full, b_conv_r, W_conv)
  a2 = agg_k(y2, src_all, dst_all)
  y3 = _tc_mid(a2, y2, deg_full, b_conv_r, w_dec)
  a3 = agg_k(y3, src_all, dst_all)
  out = _tc_out(a3, y3, deg_full, b_dec_p)
  return out[:N_NODES, :40]


# speedup vs baseline: 7.8872x; 7.8872x over previous
import functools

import jax
import jax.numpy as jnp
from jax import lax
from jax.experimental import pallas as pl
from jax.experimental.pallas import tpu as pltpu
from jax.experimental.pallas import tpu_sc as plsc

N_NODES = 10000
N_EDGES = 320000
N_PAD = 10240
H = N_PAD // 2
ROW_BLK = 2560
NC = 2
NS = 16
NW = NC * NS
CHUNK = 128
NRING = 2

ECHUNKS = 160
E_PAD = NS * ECHUNKS * CHUNK
DEG_CHUNK = 128
DEG_CHUNKS = 79
DEG_E_PAD = NW * DEG_CHUNKS * DEG_CHUNK
DEG_TRASH = N_PAD - 1

ROWS_PER_TILE = N_PAD // NS
HROWS_PER_TILE = H // NS


def _sc_mesh():
  return plsc.VectorSubcoreMesh(core_axis_name="c", subcore_axis_name="s")


def _deg_kernel_body(dst_hbm, out_hbm, dst_v, ones_v, zero_v, sem, deg_sh):
  c = lax.axis_index("c")
  s = lax.axis_index("s")
  wid = s * NC + c
  pltpu.sync_copy(dst_hbm.at[wid], dst_v)

  def fill(i, _):
    ones_v[i, :] = jnp.ones((16,), jnp.float32)
    return 0
  lax.fori_loop(0, DEG_CHUNK, fill, 0)

  def zfill(i, _):
    zero_v[i, :] = jnp.zeros((16,), jnp.float32)
    return 0
  lax.fori_loop(0, ROWS_PER_TILE, zfill, 0)

  nslice = pl.ds(s * ROWS_PER_TILE, ROWS_PER_TILE)
  pltpu.sync_copy(zero_v, deg_sh.at[nslice])
  plsc.subcore_barrier()

  def body(j, _):
    pltpu.async_copy(ones_v, deg_sh.at[dst_v.at[j]], sem, add=True).wait()
    return 0
  lax.fori_loop(0, DEG_CHUNKS, body, 0)

  plsc.subcore_barrier()
  pltpu.sync_copy(deg_sh.at[nslice], zero_v)
  pltpu.sync_copy(
      zero_v,
      out_hbm.at[pl.ds(c * N_PAD + s * ROWS_PER_TILE, ROWS_PER_TILE)],
  )


@functools.cache
def _deg_kernel():
  return pl.kernel(
      _deg_kernel_body,
      out_type=jax.ShapeDtypeStruct((NC * N_PAD, 16), jnp.float32),
      mesh=_sc_mesh(),
      scratch_types=[
          pltpu.VMEM((DEG_CHUNKS, DEG_CHUNK), jnp.int32),
          pltpu.VMEM((DEG_CHUNK, 16), jnp.float32),
          pltpu.VMEM((ROWS_PER_TILE, 16), jnp.float32),
          pltpu.SemaphoreType.DMA,
          pltpu.VMEM_SHARED((N_PAD, 16), jnp.float32),
      ],
  )


def _agg_kernel_body(y_hbm, src_hbm, dst_hbm, out_hbm,
                     src_v, dst_v, rows_v, zero_v, gsem, ssem, agg_sh):
  c = lax.axis_index("c")
  s = lax.axis_index("s")
  pltpu.sync_copy(src_hbm.at[c, s], src_v)
  pltpu.sync_copy(dst_hbm.at[c, s], dst_v)

  def zfill(i, _):
    for k in range(8):
      zero_v[i, pl.ds(k * 16, 16)] = jnp.zeros((16,), jnp.float32)
    return 0
  lax.fori_loop(0, HROWS_PER_TILE // 4, zfill, 0)

  for k in range(4):
    pltpu.sync_copy(
        zero_v,
        agg_sh.at[pl.ds(s * HROWS_PER_TILE + k * (HROWS_PER_TILE // 4),
                        HROWS_PER_TILE // 4)],
    )

  pltpu.sync_copy(zero_v.at[pl.ds(0, 8)], agg_sh.at[pl.ds(H + s * 8, 8)])
  plsc.subcore_barrier()

  def body_t(t, _):
    ghs = []
    for b in range(NRING):
      j = t * NRING + b
      ghs.append(pltpu.async_copy(
          y_hbm.at[src_v.at[j]], rows_v.at[b], gsem))
    for gh in ghs:
      gh.wait()
    shs = []
    for b in range(NRING):
      j = t * NRING + b
      shs.append(pltpu.async_copy(
          rows_v.at[b], agg_sh.at[dst_v.at[j]], ssem, add=True))
    for sh in shs:
      sh.wait()
    return 0

  lax.fori_loop(0, ECHUNKS // NRING, body_t, 0)

  plsc.subcore_barrier()
  pltpu.sync_copy(
      agg_sh.at[pl.ds(s * HROWS_PER_TILE, HROWS_PER_TILE)],
      out_hbm.at[pl.ds(c * H + s * HROWS_PER_TILE, HROWS_PER_TILE)])


@functools.cache
def _agg_kernel():
  return pl.kernel(
      _agg_kernel_body,
      out_type=jax.ShapeDtypeStruct((N_PAD, 128), jnp.float32),
      mesh=_sc_mesh(),
      scratch_types=[
          pltpu.VMEM((ECHUNKS, CHUNK), jnp.int32),
          pltpu.VMEM((ECHUNKS, CHUNK), jnp.int32),
          pltpu.VMEM((NRING, CHUNK, 128), jnp.float32),
          pltpu.VMEM((HROWS_PER_TILE // 4, 128), jnp.float32),
          pltpu.SemaphoreType.DMA,
          pltpu.SemaphoreType.DMA,
          pltpu.VMEM_SHARED((H + 128, 128), jnp.float32),
      ],
  )


def _dis_from_deg(deg_ref):
  return lax.rsqrt(deg_ref[:, 0:1] + 1.0)


def _dot(a, b):
  return jnp.dot(a, b, preferred_element_type=jnp.float32,
                 precision=lax.Precision.HIGHEST)


def _enc_body(x_ref, deg_ref, w_ref, y_ref):
  dis = _dis_from_deg(deg_ref)
  y_ref[...] = dis * _dot(x_ref[...], w_ref[...])


_tc_enc = pl.pallas_call(
    _enc_body,
    grid=(N_PAD // ROW_BLK,),
    in_specs=[
        pl.BlockSpec((ROW_BLK, 128), lambda i: (i, 0)),
        pl.BlockSpec((ROW_BLK, 128), lambda i: (i, 0)),
        pl.BlockSpec((128, 128), lambda i: (0, 0)),
    ],
    out_specs=pl.BlockSpec((ROW_BLK, 128), lambda i: (i, 0)),
    out_shape=jax.ShapeDtypeStruct((N_PAD, 128), jnp.float32),
)


def _mid_body(a_ref, y_ref, deg_ref, b_ref, w_ref, o_ref):
  dis = _dis_from_deg(deg_ref)
  h = jnp.maximum(dis * (a_ref[...] + y_ref[...]) + b_ref[...], 0.0)
  o_ref[...] = dis * _dot(h, w_ref[...])


_tc_mid = pl.pallas_call(
    _mid_body,
    grid=(N_PAD // ROW_BLK,),
    in_specs=[
        pl.BlockSpec((ROW_BLK, 128), lambda i: (i, 0)),
        pl.BlockSpec((ROW_BLK, 128), lambda i: (i, 0)),
        pl.BlockSpec((ROW_BLK, 128), lambda i: (i, 0)),
        pl.BlockSpec((1, 128), lambda i: (0, 0)),
        pl.BlockSpec((128, 128), lambda i: (0, 0)),
    ],
    out_specs=pl.BlockSpec((ROW_BLK, 128), lambda i: (i, 0)),
    out_shape=jax.ShapeDtypeStruct((N_PAD, 128), jnp.float32),
)


def _out_body(a_ref, y_ref, deg_ref, b_ref, o_ref):
  dis = _dis_from_deg(deg_ref)
  o_ref[...] = jnp.maximum(dis * (a_ref[...] + y_ref[...]) + b_ref[...], 0.0)


_tc_out = pl.pallas_call(
    _out_body,
    grid=(N_PAD // ROW_BLK,),
    in_specs=[
        pl.BlockSpec((ROW_BLK, 128), lambda i: (i, 0)),
        pl.BlockSpec((ROW_BLK, 128), lambda i: (i, 0)),
        pl.BlockSpec((ROW_BLK, 128), lambda i: (i, 0)),
        pl.BlockSpec((1, 128), lambda i: (0, 0)),
    ],
    out_specs=pl.BlockSpec((ROW_BLK, 128), lambda i: (i, 0)),
    out_shape=jax.ShapeDtypeStruct((N_PAD, 128), jnp.float32),
)


@jax.jit
def kernel(x, edge_index, W_enc, b_enc, W_conv, b_conv, W_dec, b_dec):
  src = edge_index[0]
  dst = edge_index[1]

  pad = E_PAD - N_EDGES
  src_p = jnp.concatenate([src, jnp.zeros((pad,), jnp.int32)])
  dst_p = jnp.concatenate([dst, jnp.full((pad,), -1, jnp.int32)])
  src_all = jnp.stack([src_p, src_p]).reshape(NC, NS, ECHUNKS, CHUNK)
  deg_src = (jnp.arange(E_PAD, dtype=jnp.int32) % N_PAD)
  deg_src = jnp.stack([deg_src, deg_src]).reshape(NC, NS, ECHUNKS, CHUNK)
  trash = H + (dst_p & 127)
  dst_cs = []
  for c in range(NC):
    in_c = (dst_p >= c * H) & (dst_p < (c + 1) * H)
    dst_cs.append(jnp.where(in_c, dst_p - c * H, trash))
  dst_all = jnp.stack(dst_cs).reshape(NC, NS, ECHUNKS, CHUNK)

  x_pad = jnp.pad(x, ((0, N_PAD - N_NODES), (0, 0)))
  w_dec = jnp.pad(W_dec, ((0, 0), (0, 128 - W_dec.shape[1])))
  b_dec_p = jnp.pad(b_dec, (0, 128 - b_dec.shape[0])).reshape(1, 128)
  b_enc_r = b_enc.reshape(1, 128)
  b_conv_r = b_conv.reshape(1, 128)

  agg_k = _agg_kernel()
  ones_t = jnp.ones((N_PAD, 128), jnp.float32)
  deg_full = agg_k(ones_t, deg_src, dst_all)

  y0 = _tc_enc(x_pad, deg_full, W_enc)
  a0 = agg_k(y0, src_all, dst_all)
  y1 = _tc_mid(a0, y0, deg_full, b_enc_r, W_conv)
  a1 = agg_k(y1, src_all, dst_all)
  y2 = _tc_mid(a1, y1, deg_full, b_conv_r, W_conv)
  a2 = agg_k(y2, src_all, dst_all)
  y3 = _tc_mid(a2, y2, deg_full, b_conv_r, w_dec)
  a3 = agg_k(y3, src_all, dst_all)
  out = _tc_out(a3, y3, deg_full, b_dec_p)
  return out[:N_NODES, :40]


# speedup vs baseline: 8.6093x; 1.0916x over previous
import functools

import jax
import jax.numpy as jnp
from jax import lax
from jax.experimental import pallas as pl
from jax.experimental.pallas import tpu as pltpu
from jax.experimental.pallas import tpu_sc as plsc

N_NODES = 10000
N_EDGES = 320000
N_PAD = 10240
H = N_PAD // 2
ROW_BLK = 2560
NC = 2
NS = 16
NW = NC * NS
CHUNK = 128
NRING = 2

ECHUNKS = 160
E_PAD = NS * ECHUNKS * CHUNK
DEG_CHUNK = 128
DEG_CHUNKS = 79
DEG_E_PAD = NW * DEG_CHUNKS * DEG_CHUNK
DEG_TRASH = N_PAD - 1

ROWS_PER_TILE = N_PAD // NS
HROWS_PER_TILE = H // NS


def _sc_mesh():
  return plsc.VectorSubcoreMesh(core_axis_name="c", subcore_axis_name="s")


def _deg_kernel_body(dst_hbm, out_hbm, dst_v, ones_v, zero_v, sem, deg_sh):
  c = lax.axis_index("c")
  s = lax.axis_index("s")
  wid = s * NC + c
  pltpu.sync_copy(dst_hbm.at[wid], dst_v)

  def fill(i, _):
    ones_v[i, :] = jnp.ones((16,), jnp.float32)
    return 0
  lax.fori_loop(0, DEG_CHUNK, fill, 0)

  def zfill(i, _):
    zero_v[i, :] = jnp.zeros((16,), jnp.float32)
    return 0
  lax.fori_loop(0, ROWS_PER_TILE, zfill, 0)

  nslice = pl.ds(s * ROWS_PER_TILE, ROWS_PER_TILE)
  pltpu.sync_copy(zero_v, deg_sh.at[nslice])
  plsc.subcore_barrier()

  def body(j, _):
    pltpu.async_copy(ones_v, deg_sh.at[dst_v.at[j]], sem, add=True).wait()
    return 0
  lax.fori_loop(0, DEG_CHUNKS, body, 0)

  plsc.subcore_barrier()
  pltpu.sync_copy(deg_sh.at[nslice], zero_v)
  pltpu.sync_copy(
      zero_v,
      out_hbm.at[pl.ds(c * N_PAD + s * ROWS_PER_TILE, ROWS_PER_TILE)],
  )


@functools.cache
def _deg_kernel():
  return pl.kernel(
      _deg_kernel_body,
      out_type=jax.ShapeDtypeStruct((NC * N_PAD, 16), jnp.float32),
      mesh=_sc_mesh(),
      scratch_types=[
          pltpu.VMEM((DEG_CHUNKS, DEG_CHUNK), jnp.int32),
          pltpu.VMEM((DEG_CHUNK, 16), jnp.float32),
          pltpu.VMEM((ROWS_PER_TILE, 16), jnp.float32),
          pltpu.SemaphoreType.DMA,
          pltpu.VMEM_SHARED((N_PAD, 16), jnp.float32),
      ],
  )


def _agg_kernel_body(y_hbm, src_hbm, dst_hbm, out_hbm,
                     src_v, dst_v, rows_v, zero_v, gsem, ssem, agg_sh):
  c = lax.axis_index("c")
  s = lax.axis_index("s")
  pltpu.sync_copy(src_hbm.at[c, s], src_v)
  pltpu.sync_copy(dst_hbm.at[c, s], dst_v)

  def zfill(i, _):
    for k in range(8):
      zero_v[i, pl.ds(k * 16, 16)] = jnp.zeros((16,), jnp.float32)
    return 0
  lax.fori_loop(0, HROWS_PER_TILE // 4, zfill, 0)

  for k in range(4):
    pltpu.sync_copy(
        zero_v,
        agg_sh.at[pl.ds(s * HROWS_PER_TILE + k * (HROWS_PER_TILE // 4),
                        HROWS_PER_TILE // 4)],
    )

  pltpu.sync_copy(zero_v.at[pl.ds(0, 8)], agg_sh.at[pl.ds(H + s * 8, 8)])
  plsc.subcore_barrier()

  def _gather(j, rb):
    pltpu.async_copy(y_hbm.at[src_v.at[j]], rows_v.at[rb], gsem)

  def _scatter(j, rb):
    pltpu.async_copy(rows_v.at[rb], agg_sh.at[dst_v.at[j]], ssem, add=True)

  def _wait_gather(j, rb):
    pltpu.make_async_copy(y_hbm.at[src_v.at[j]], rows_v.at[rb], gsem).wait()

  def _wait_scatter(j, rb):
    pltpu.make_async_copy(rows_v.at[rb], agg_sh.at[dst_v.at[j]], ssem).wait()

  _gather(0, 0)

  def body_t(t, _):
    rb = lax.rem(t, 2)

    @pl.when(t >= 2)
    def _():
      _wait_scatter(t - 2, rb)

    _gather(t, rb)
    _wait_gather(t - 1, 1 - rb)
    _scatter(t - 1, 1 - rb)
    return 0

  lax.fori_loop(1, ECHUNKS, body_t, 0)
  last = ECHUNKS - 1
  _wait_gather(last, last % 2)
  _scatter(last, last % 2)
  _wait_scatter(last - 1, (last - 1) % 2)
  _wait_scatter(last, last % 2)

  plsc.subcore_barrier()
  pltpu.sync_copy(
      agg_sh.at[pl.ds(s * HROWS_PER_TILE, HROWS_PER_TILE)],
      out_hbm.at[pl.ds(c * H + s * HROWS_PER_TILE, HROWS_PER_TILE)])


@functools.cache
def _agg_kernel():
  return pl.kernel(
      _agg_kernel_body,
      out_type=jax.ShapeDtypeStruct((N_PAD, 128), jnp.float32),
      mesh=_sc_mesh(),
      scratch_types=[
          pltpu.VMEM((ECHUNKS, CHUNK), jnp.int32),
          pltpu.VMEM((ECHUNKS, CHUNK), jnp.int32),
          pltpu.VMEM((NRING, CHUNK, 128), jnp.float32),
          pltpu.VMEM((HROWS_PER_TILE // 4, 128), jnp.float32),
          pltpu.SemaphoreType.DMA,
          pltpu.SemaphoreType.DMA,
          pltpu.VMEM_SHARED((H + 128, 128), jnp.float32),
      ],
  )


def _dis_from_deg(deg_ref):
  return lax.rsqrt(deg_ref[:, 0:1] + 1.0)


def _dot(a, b):
  return jnp.dot(a, b, preferred_element_type=jnp.float32,
                 precision=lax.Precision.HIGHEST)


def _enc_body(x_ref, deg_ref, w_ref, y_ref):
  dis = _dis_from_deg(deg_ref)
  y_ref[...] = dis * _dot(x_ref[...], w_ref[...])


_tc_enc = pl.pallas_call(
    _enc_body,
    grid=(N_PAD // ROW_BLK,),
    in_specs=[
        pl.BlockSpec((ROW_BLK, 128), lambda i: (i, 0)),
        pl.BlockSpec((ROW_BLK, 128), lambda i: (i, 0)),
        pl.BlockSpec((128, 128), lambda i: (0, 0)),
    ],
    out_specs=pl.BlockSpec((ROW_BLK, 128), lambda i: (i, 0)),
    out_shape=jax.ShapeDtypeStruct((N_PAD, 128), jnp.float32),
)


def _mid_body(a_ref, y_ref, deg_ref, b_ref, w_ref, o_ref):
  dis = _dis_from_deg(deg_ref)
  h = jnp.maximum(dis * (a_ref[...] + y_ref[...]) + b_ref[...], 0.0)
  o_ref[...] = dis * _dot(h, w_ref[...])


_tc_mid = pl.pallas_call(
    _mid_body,
    grid=(N_PAD // ROW_BLK,),
    in_specs=[
        pl.BlockSpec((ROW_BLK, 128), lambda i: (i, 0)),
        pl.BlockSpec((ROW_BLK, 128), lambda i: (i, 0)),
        pl.BlockSpec((ROW_BLK, 128), lambda i: (i, 0)),
        pl.BlockSpec((1, 128), lambda i: (0, 0)),
        pl.BlockSpec((128, 128), lambda i: (0, 0)),
    ],
    out_specs=pl.BlockSpec((ROW_BLK, 128), lambda i: (i, 0)),
    out_shape=jax.ShapeDtypeStruct((N_PAD, 128), jnp.float32),
)


def _out_body(a_ref, y_ref, deg_ref, b_ref, o_ref):
  dis = _dis_from_deg(deg_ref)
  o_ref[...] = jnp.maximum(dis * (a_ref[...] + y_ref[...]) + b_ref[...], 0.0)


_tc_out = pl.pallas_call(
    _out_body,
    grid=(N_PAD // ROW_BLK,),
    in_specs=[
        pl.BlockSpec((ROW_BLK, 128), lambda i: (i, 0)),
        pl.BlockSpec((ROW_BLK, 128), lambda i: (i, 0)),
        pl.BlockSpec((ROW_BLK, 128), lambda i: (i, 0)),
        pl.BlockSpec((1, 128), lambda i: (0, 0)),
    ],
    out_specs=pl.BlockSpec((ROW_BLK, 128), lambda i: (i, 0)),
    out_shape=jax.ShapeDtypeStruct((N_PAD, 128), jnp.float32),
)


@jax.jit
def kernel(x, edge_index, W_enc, b_enc, W_conv, b_conv, W_dec, b_dec):
  src = edge_index[0]
  dst = edge_index[1]

  pad = E_PAD - N_EDGES
  src_p = jnp.concatenate([src, jnp.zeros((pad,), jnp.int32)])
  dst_p = jnp.concatenate([dst, jnp.full((pad,), -1, jnp.int32)])
  src_all = jnp.stack([src_p, src_p]).reshape(NC, NS, ECHUNKS, CHUNK)
  deg_src = (jnp.arange(E_PAD, dtype=jnp.int32) % N_PAD)
  deg_src = jnp.stack([deg_src, deg_src]).reshape(NC, NS, ECHUNKS, CHUNK)
  trash = H + (dst_p & 127)
  dst_cs = []
  for c in range(NC):
    in_c = (dst_p >= c * H) & (dst_p < (c + 1) * H)
    dst_cs.append(jnp.where(in_c, dst_p - c * H, trash))
  dst_all = jnp.stack(dst_cs).reshape(NC, NS, ECHUNKS, CHUNK)

  x_pad = jnp.pad(x, ((0, N_PAD - N_NODES), (0, 0)))
  w_dec = jnp.pad(W_dec, ((0, 0), (0, 128 - W_dec.shape[1])))
  b_dec_p = jnp.pad(b_dec, (0, 128 - b_dec.shape[0])).reshape(1, 128)
  b_enc_r = b_enc.reshape(1, 128)
  b_conv_r = b_conv.reshape(1, 128)

  agg_k = _agg_kernel()
  ones_t = jnp.ones((N_PAD, 128), jnp.float32)
  deg_full = agg_k(ones_t, deg_src, dst_all)

  y0 = _tc_enc(x_pad, deg_full, W_enc)
  a0 = agg_k(y0, src_all, dst_all)
  y1 = _tc_mid(a0, y0, deg_full, b_enc_r, W_conv)
  a1 = agg_k(y1, src_all, dst_all)
  y2 = _tc_mid(a1, y1, deg_full, b_conv_r, W_conv)
  a2 = agg_k(y2, src_all, dst_all)
  y3 = _tc_mid(a2, y2, deg_full, b_conv_r, w_dec)
  a3 = agg_k(y3, src_all, dst_all)
  out = _tc_out(a3, y3, deg_full, b_dec_p)
  return out[:N_NODES, :40]
